# Initial kernel scaffold; baseline (speedup 1.0000x reference)
import functools

import jax
import jax.numpy as jnp
from jax import lax
from jax.experimental import pallas as pl
from jax.experimental.pallas import tpu as pltpu
from jax.experimental.pallas import tpu_sc as plsc

N = 10000
D = 128
E = 320000
NC = 2
NS = 16
NW = NC * NS
EPT = E // NW
C = 80
NCHUNK = EPT // C
RPT = N // NS
DEGW = 16

_f32 = jnp.float32
_mesh = plsc.VectorSubcoreMesh(core_axis_name="c", subcore_axis_name="s")


def _make_agg_kernel(with_deg):
    out_types = [jax.ShapeDtypeStruct((NC, N, D), _f32)]
    scratch = [
        pltpu.VMEM_SHARED((N, D), _f32),
        pltpu.VMEM((C,), jnp.int32),
        pltpu.VMEM((C,), jnp.int32),
        pltpu.VMEM((C, D), _f32),
    ]
    if with_deg:
        out_types.append(jax.ShapeDtypeStruct((NC, N, DEGW), _f32))
        scratch.append(pltpu.VMEM_SHARED((N, DEGW), _f32))
        scratch.append(pltpu.VMEM((C, DEGW), _f32))

    def body(*refs):
        if with_deg:
            (mat_hbm, src_hbm, dst_hbm, zrow_hbm, zdeg_hbm, ones_hbm,
             out_hbm, deg_hbm, acc_sh, src_v, dst_v, rows_v, deg_sh,
             ones_v) = refs
        else:
            (mat_hbm, src_hbm, dst_hbm, zrow_hbm,
             out_hbm, acc_sh, src_v, dst_v, rows_v) = refs
        c = lax.axis_index("c")
        s = lax.axis_index("s")
        wid = c * NS + s
        base = wid * EPT
        r0 = s * RPT

        pltpu.sync_copy(zrow_hbm.at[pl.ds(r0, RPT)], acc_sh.at[pl.ds(r0, RPT)])
        if with_deg:
            pltpu.sync_copy(zdeg_hbm.at[pl.ds(r0, RPT)],
                            deg_sh.at[pl.ds(r0, RPT)])
            pltpu.sync_copy(ones_hbm, ones_v)
        plsc.subcore_barrier()

        @pl.loop(0, NCHUNK)
        def _(i):
            eb = base + i * C
            pltpu.sync_copy(src_hbm.at[pl.ds(eb, C)], src_v)
            pltpu.sync_copy(dst_hbm.at[pl.ds(eb, C)], dst_v)
            pltpu.sync_copy(mat_hbm.at[src_v], rows_v)
            pltpu.sync_copy(rows_v, acc_sh.at[dst_v], add=True)
            if with_deg:
                pltpu.sync_copy(ones_v, deg_sh.at[dst_v], add=True)

        plsc.subcore_barrier()
        pltpu.sync_copy(acc_sh.at[pl.ds(r0, RPT)],
                        out_hbm.at[c, pl.ds(r0, RPT)])
        if with_deg:
            pltpu.sync_copy(deg_sh.at[pl.ds(r0, RPT)],
                            deg_hbm.at[c, pl.ds(r0, RPT)])

    return pl.kernel(
        body,
        out_type=tuple(out_types) if with_deg else out_types[0],
        mesh=_mesh,
        scratch_types=scratch,
    )


_agg_deg = _make_agg_kernel(with_deg=True)
_agg = _make_agg_kernel(with_deg=False)


_RB = 1000


def _mm_body(x_ref, w_ref, o_ref):
    o_ref[...] = jnp.dot(x_ref[...], w_ref[...],
                         preferred_element_type=_f32,
                         precision=lax.Precision.HIGHEST)


def _matmul(x, w):
    return pl.pallas_call(
        _mm_body,
        grid=(N // _RB,),
        in_specs=[pl.BlockSpec((_RB, D), lambda i: (i, 0)),
                  pl.BlockSpec((D, D), lambda i: (0, 0))],
        out_specs=pl.BlockSpec((_RB, D), lambda i: (i, 0)),
        out_shape=jax.ShapeDtypeStruct((N, D), _f32),
    )(x, w)


def _mid_body(part_ref, deg_ref, w_ref, b0_ref, y_ref):
    h = part_ref[0] + part_ref[1] + b0_ref[...]
    xw1 = jnp.dot(h, w_ref[...], preferred_element_type=_f32,
                  precision=lax.Precision.HIGHEST)
    deg = deg_ref[0, :, 0:1] + deg_ref[1, :, 0:1] + 1.0
    y_ref[...] = lax.rsqrt(deg) * xw1


def _mid(part, deg, w1, b0):
    return pl.pallas_call(
        _mid_body,
        grid=(N // _RB,),
        in_specs=[pl.BlockSpec((NC, _RB, D), lambda i: (0, i, 0)),
                  pl.BlockSpec((NC, _RB, DEGW), lambda i: (0, i, 0)),
                  pl.BlockSpec((D, D), lambda i: (0, 0)),
                  pl.BlockSpec((1, D), lambda i: (0, 0))],
        out_specs=pl.BlockSpec((_RB, D), lambda i: (i, 0)),
        out_shape=jax.ShapeDtypeStruct((N, D), _f32),
    )(part, deg, w1, b0)


def _fin_body(part_ref, deg_ref, y_ref, b1_ref, o_ref):
    agg = part_ref[0] + part_ref[1] + y_ref[...]
    deg = deg_ref[0, :, 0:1] + deg_ref[1, :, 0:1] + 1.0
    o_ref[...] = lax.rsqrt(deg) * agg + b1_ref[...]


def _final(part, deg, y, b1):
    return pl.pallas_call(
        _fin_body,
        grid=(N // _RB,),
        in_specs=[pl.BlockSpec((NC, _RB, D), lambda i: (0, i, 0)),
                  pl.BlockSpec((NC, _RB, DEGW), lambda i: (0, i, 0)),
                  pl.BlockSpec((_RB, D), lambda i: (i, 0)),
                  pl.BlockSpec((1, D), lambda i: (0, 0))],
        out_specs=pl.BlockSpec((_RB, D), lambda i: (i, 0)),
        out_shape=jax.ShapeDtypeStruct((N, D), _f32),
    )(part, deg, y, b1)


def kernel(edge_index, x_init, W0, b0, W1, b1):
    src = edge_index[0].astype(jnp.int32)
    dst = edge_index[1].astype(jnp.int32)
    zrow = jnp.zeros((N, D), _f32)
    zdeg = jnp.zeros((N, DEGW), _f32)
    ones = jnp.ones((C, DEGW), _f32)
    b0r = b0.reshape(1, D)
    b1r = b1.reshape(1, D)

    xw0 = _matmul(x_init, W0)
    part0, deg = _agg_deg(xw0, src, dst, zrow, zdeg, ones)
    y = _mid(part0, deg, W1, b0r)
    part1 = _agg(y, src, dst, zrow)
    return _final(part1, deg, y, b1r)


# speedup vs baseline: 11.3493x; 11.3493x over previous
import functools

import jax
import jax.numpy as jnp
from jax import lax
from jax.experimental import pallas as pl
from jax.experimental.pallas import tpu as pltpu
from jax.experimental.pallas import tpu_sc as plsc

N = 10000
NP = 10240
D = 128
E = 320000
NC = 2
NS = 16
NW = NC * NS
EPT = E // NW
C = 80
NCHUNK = EPT // C
RPT = NP // NS
DEGW = 16

_f32 = jnp.float32
_mesh = plsc.VectorSubcoreMesh(core_axis_name="c", subcore_axis_name="s")


def _make_agg_kernel():
    def body(mat_hbm, src_hbm, dst_hbm, zrow_hbm, out_hbm,
             acc_sh, src_v, dst_v, rows_v):
        c = lax.axis_index("c")
        s = lax.axis_index("s")
        wid = c * NS + s
        base = wid * EPT
        r0 = s * RPT
        nslab = RPT // C

        pltpu.sync_copy(zrow_hbm.at[pl.ds(0, C)], rows_v)

        @pl.loop(0, nslab)
        def _(k):
            pltpu.sync_copy(rows_v, acc_sh.at[pl.ds(r0 + k * C, C)])

        plsc.subcore_barrier()

        @pl.loop(0, NCHUNK)
        def _(i):
            eb = base + i * C
            pltpu.sync_copy(src_hbm.at[pl.ds(eb, C)], src_v)
            pltpu.sync_copy(dst_hbm.at[pl.ds(eb, C)], dst_v)
            pltpu.sync_copy(mat_hbm.at[src_v], rows_v)
            pltpu.sync_copy(rows_v, acc_sh.at[dst_v], add=True)

        plsc.subcore_barrier()
        ob = c * NP + r0

        @pl.loop(0, nslab)
        def _(k):
            pltpu.sync_copy(acc_sh.at[pl.ds(r0 + k * C, C)], rows_v)
            pltpu.sync_copy(rows_v, out_hbm.at[pl.ds(ob + k * C, C)])

    return pl.kernel(
        body,
        out_type=jax.ShapeDtypeStruct((NC * NP, D), _f32),
        mesh=_mesh,
        scratch_types=[
            pltpu.VMEM_SHARED((NP, D), _f32),
            pltpu.VMEM((C,), jnp.int32),
            pltpu.VMEM((C,), jnp.int32),
            pltpu.VMEM((C, D), _f32),
        ],
    )


def _make_deg_kernel():
    def body(dst_hbm, zdeg_hbm, ones_hbm, deg_hbm, deg_sh, dst_v, ones_v):
        c = lax.axis_index("c")
        s = lax.axis_index("s")
        wid = c * NS + s
        base = wid * EPT
        r0 = s * RPT
        nslab = RPT // C

        pltpu.sync_copy(zdeg_hbm.at[pl.ds(0, C)], ones_v)

        @pl.loop(0, nslab)
        def _(k):
            pltpu.sync_copy(ones_v, deg_sh.at[pl.ds(r0 + k * C, C)])

        pltpu.sync_copy(ones_hbm, ones_v)
        plsc.subcore_barrier()

        @pl.loop(0, NCHUNK)
        def _(i):
            pltpu.sync_copy(dst_hbm.at[pl.ds(base + i * C, C)], dst_v)
            pltpu.sync_copy(ones_v, deg_sh.at[dst_v], add=True)

        plsc.subcore_barrier()
        ob = c * NP + r0

        @pl.loop(0, nslab)
        def _(k):
            pltpu.sync_copy(deg_sh.at[pl.ds(r0 + k * C, C)], ones_v)
            pltpu.sync_copy(ones_v, deg_hbm.at[pl.ds(ob + k * C, C)])

    return pl.kernel(
        body,
        out_type=jax.ShapeDtypeStruct((NC * NP, DEGW), _f32),
        mesh=_mesh,
        scratch_types=[
            pltpu.VMEM_SHARED((NP, DEGW), _f32),
            pltpu.VMEM((C,), jnp.int32),
            pltpu.VMEM((C, DEGW), _f32),
        ],
    )


_agg = _make_agg_kernel()
_deg = _make_deg_kernel()


_RB = 1280


def _mm_body(x_ref, w_ref, o_ref):
    o_ref[...] = jnp.dot(x_ref[...], w_ref[...],
                         preferred_element_type=_f32,
                         precision=lax.Precision.HIGHEST)


def _matmul(x, w):
    return pl.pallas_call(
        _mm_body,
        grid=(NP // _RB,),
        in_specs=[pl.BlockSpec((_RB, D), lambda i: (i, 0)),
                  pl.BlockSpec((D, D), lambda i: (0, 0))],
        out_specs=pl.BlockSpec((_RB, D), lambda i: (i, 0)),
        out_shape=jax.ShapeDtypeStruct((NP, D), _f32),
    )(x, w)


def _mid_body(part_ref, deg_ref, w_ref, b0_ref, y_ref):
    h = part_ref[0] + part_ref[1] + b0_ref[...]
    xw1 = jnp.dot(h, w_ref[...], preferred_element_type=_f32,
                  precision=lax.Precision.HIGHEST)
    deg = deg_ref[0, :, 0:1] + deg_ref[1, :, 0:1] + 1.0
    y_ref[...] = lax.rsqrt(deg) * xw1


def _mid(part, deg, w1, b0):
    return pl.pallas_call(
        _mid_body,
        grid=(NP // _RB,),
        in_specs=[pl.BlockSpec((NC, _RB, D), lambda i: (0, i, 0)),
                  pl.BlockSpec((NC, _RB, DEGW), lambda i: (0, i, 0)),
                  pl.BlockSpec((D, D), lambda i: (0, 0)),
                  pl.BlockSpec((1, D), lambda i: (0, 0))],
        out_specs=pl.BlockSpec((_RB, D), lambda i: (i, 0)),
        out_shape=jax.ShapeDtypeStruct((NP, D), _f32),
    )(part, deg, w1, b0)


def _fin_body(part_ref, deg_ref, y_ref, b1_ref, o_ref):
    agg = part_ref[0] + part_ref[1] + y_ref[...]
    deg = deg_ref[0, :, 0:1] + deg_ref[1, :, 0:1] + 1.0
    o_ref[...] = lax.rsqrt(deg) * agg + b1_ref[...]


def _final(part, deg, y, b1):
    return pl.pallas_call(
        _fin_body,
        grid=(NP // _RB,),
        in_specs=[pl.BlockSpec((NC, _RB, D), lambda i: (0, i, 0)),
                  pl.BlockSpec((NC, _RB, DEGW), lambda i: (0, i, 0)),
                  pl.BlockSpec((_RB, D), lambda i: (i, 0)),
                  pl.BlockSpec((1, D), lambda i: (0, 0))],
        out_specs=pl.BlockSpec((_RB, D), lambda i: (i, 0)),
        out_shape=jax.ShapeDtypeStruct((NP, D), _f32),
    )(part, deg, y, b1)


def kernel(edge_index, x_init, W0, b0, W1, b1):
    src = edge_index[0].astype(jnp.int32)
    dst = edge_index[1].astype(jnp.int32)
    xp = jnp.zeros((NP, D), _f32).at[:N].set(x_init)
    zrow = jnp.zeros((NP, D), _f32)
    zdeg = jnp.zeros((NP, DEGW), _f32)
    ones = jnp.ones((C, DEGW), _f32)
    b0r = b0.reshape(1, D)
    b1r = b1.reshape(1, D)

    deg = _deg(dst, zdeg, ones).reshape(NC, NP, DEGW)
    xw0 = _matmul(xp, W0)
    part0 = _agg(xw0, src, dst, zrow).reshape(NC, NP, D)
    y = _mid(part0, deg, W1, b0r)
    part1 = _agg(y, src, dst, zrow).reshape(NC, NP, D)
    return _final(part1, deg, y, b1r)[:N]


# speedup vs baseline: 18.7755x; 1.6543x over previous
import functools

import jax
import jax.numpy as jnp
from jax import lax
from jax.experimental import pallas as pl
from jax.experimental.pallas import tpu as pltpu
from jax.experimental.pallas import tpu_sc as plsc

N = 10000
NP = 10240
D = 128
E = 320000
NC = 2
NS = 16
NW = NC * NS
EPT = E // NW
C = 80
NCHUNK = EPT // C
RPT = NP // NS
DEGW = 16

_f32 = jnp.float32
_mesh = plsc.VectorSubcoreMesh(core_axis_name="c", subcore_axis_name="s")


def _make_agg_kernel():
    def body(mat_hbm, src_hbm, dst_hbm, zrow_hbm, out_hbm, acc_sh,
             sv0, sv1, sv2, sv3, dv0, dv1, dv2, dv3, rows0, rows1,
             isem0, isem1, isem2, isem3, gsem0, gsem1, ssem0, ssem1):
        c = lax.axis_index("c")
        s = lax.axis_index("s")
        wid = c * NS + s
        base = wid * EPT
        r0 = s * RPT
        nslab = RPT // C
        sv = [sv0, sv1, sv2, sv3]
        dv = [dv0, dv1, dv2, dv3]
        rows = [rows0, rows1]
        isem = [isem0, isem1, isem2, isem3]
        gsem = [gsem0, gsem1]
        ssem = [ssem0, ssem1]

        def issue_idx(i, k):
            eb = base + i * C
            pltpu.async_copy(src_hbm.at[pl.ds(eb, C)], sv[k], isem[k])
            pltpu.async_copy(dst_hbm.at[pl.ds(eb, C)], dv[k], isem[k])

        def wait_idx(k):
            pltpu.make_async_copy(src_hbm.at[pl.ds(0, C)], sv[k],
                                  isem[k]).wait()
            pltpu.make_async_copy(dst_hbm.at[pl.ds(0, C)], dv[k],
                                  isem[k]).wait()

        def issue_gather(k, b):
            pltpu.async_copy(mat_hbm.at[sv[k]], rows[b], gsem[b])

        def wait_gather(k, b):
            pltpu.make_async_copy(mat_hbm.at[sv[k]], rows[b],
                                  gsem[b]).wait()

        def issue_scatter(k, b):
            pltpu.async_copy(rows[b], acc_sh.at[dv[k]], ssem[b], add=True)

        def wait_scatter(k, b):
            pltpu.make_async_copy(rows[b], acc_sh.at[dv[k]],
                                  ssem[b]).wait()

        pltpu.sync_copy(zrow_hbm.at[pl.ds(0, C)], rows0)

        @pl.loop(0, nslab)
        def _(k):
            pltpu.sync_copy(rows0, acc_sh.at[pl.ds(r0 + k * C, C)])

        plsc.subcore_barrier()

        issue_idx(0, 0)
        issue_idx(1, 1)
        issue_idx(2, 2)
        issue_idx(3, 3)
        wait_idx(0)
        issue_gather(0, 0)
        wait_idx(1)
        issue_gather(1, 1)
        wait_gather(0, 0)
        issue_scatter(0, 0)
        wait_gather(1, 1)
        issue_scatter(1, 1)

        def step(i, k, b, nk, prefetch):
            wait_scatter(nk, b)
            if prefetch:
                issue_idx(i + 2, nk)
            wait_idx(k)
            issue_gather(k, b)
            wait_gather(k, b)
            issue_scatter(k, b)

        @pl.loop(0, (NCHUNK - 5) // 4)
        def _(g):
            i = 2 + 4 * g
            step(i + 0, 2, 0, 0, True)
            step(i + 1, 3, 1, 1, True)
            step(i + 2, 0, 0, 2, True)
            step(i + 3, 1, 1, 3, True)

        step(NCHUNK - 3, 2, 0, 0, True)
        step(NCHUNK - 2, 3, 1, 1, False)
        step(NCHUNK - 1, 0, 0, 2, False)
        wait_scatter(3, 1)
        wait_scatter(0, 0)

        plsc.subcore_barrier()
        ob = c * NP + r0

        @pl.loop(0, nslab)
        def _(k):
            pltpu.sync_copy(acc_sh.at[pl.ds(r0 + k * C, C)], rows0)
            pltpu.sync_copy(rows0, out_hbm.at[pl.ds(ob + k * C, C)])

    return pl.kernel(
        body,
        out_type=jax.ShapeDtypeStruct((NC * NP, D), _f32),
        mesh=_mesh,
        scratch_types=(
            [pltpu.VMEM_SHARED((NP, D), _f32)]
            + [pltpu.VMEM((C,), jnp.int32) for _ in range(8)]
            + [pltpu.VMEM((C, D), _f32) for _ in range(2)]
            + [pltpu.SemaphoreType.DMA for _ in range(8)]
        ),
    )


def _make_deg_kernel():
    def body(dst_hbm, zdeg_hbm, ones_hbm, deg_hbm, deg_sh, dst_v, ones_v):
        c = lax.axis_index("c")
        s = lax.axis_index("s")
        wid = c * NS + s
        base = wid * EPT
        r0 = s * RPT
        nslab = RPT // C

        pltpu.sync_copy(zdeg_hbm.at[pl.ds(0, C)], ones_v)

        @pl.loop(0, nslab)
        def _(k):
            pltpu.sync_copy(ones_v, deg_sh.at[pl.ds(r0 + k * C, C)])

        pltpu.sync_copy(ones_hbm, ones_v)
        plsc.subcore_barrier()

        @pl.loop(0, NCHUNK)
        def _(i):
            pltpu.sync_copy(dst_hbm.at[pl.ds(base + i * C, C)], dst_v)
            pltpu.sync_copy(ones_v, deg_sh.at[dst_v], add=True)

        plsc.subcore_barrier()
        ob = c * NP + r0

        @pl.loop(0, nslab)
        def _(k):
            pltpu.sync_copy(deg_sh.at[pl.ds(r0 + k * C, C)], ones_v)
            pltpu.sync_copy(ones_v, deg_hbm.at[pl.ds(ob + k * C, C)])

    return pl.kernel(
        body,
        out_type=jax.ShapeDtypeStruct((NC * NP, DEGW), _f32),
        mesh=_mesh,
        scratch_types=[
            pltpu.VMEM_SHARED((NP, DEGW), _f32),
            pltpu.VMEM((C,), jnp.int32),
            pltpu.VMEM((C, DEGW), _f32),
        ],
    )


_agg = _make_agg_kernel()
_deg = _make_deg_kernel()


_RB = 1280


def _mm_body(x_ref, w_ref, o_ref):
    o_ref[...] = jnp.dot(x_ref[...], w_ref[...],
                         preferred_element_type=_f32,
                         precision=lax.Precision.HIGHEST)


def _matmul(x, w):
    return pl.pallas_call(
        _mm_body,
        grid=(NP // _RB,),
        in_specs=[pl.BlockSpec((_RB, D), lambda i: (i, 0)),
                  pl.BlockSpec((D, D), lambda i: (0, 0))],
        out_specs=pl.BlockSpec((_RB, D), lambda i: (i, 0)),
        out_shape=jax.ShapeDtypeStruct((NP, D), _f32),
    )(x, w)


def _mid_body(part_ref, deg_ref, w_ref, b0_ref, y_ref):
    h = part_ref[0] + part_ref[1] + b0_ref[...]
    xw1 = jnp.dot(h, w_ref[...], preferred_element_type=_f32,
                  precision=lax.Precision.HIGHEST)
    deg = deg_ref[0, :, 0:1] + deg_ref[1, :, 0:1] + 1.0
    y_ref[...] = lax.rsqrt(deg) * xw1


def _mid(part, deg, w1, b0):
    return pl.pallas_call(
        _mid_body,
        grid=(NP // _RB,),
        in_specs=[pl.BlockSpec((NC, _RB, D), lambda i: (0, i, 0)),
                  pl.BlockSpec((NC, _RB, DEGW), lambda i: (0, i, 0)),
                  pl.BlockSpec((D, D), lambda i: (0, 0)),
                  pl.BlockSpec((1, D), lambda i: (0, 0))],
        out_specs=pl.BlockSpec((_RB, D), lambda i: (i, 0)),
        out_shape=jax.ShapeDtypeStruct((NP, D), _f32),
    )(part, deg, w1, b0)


def _fin_body(part_ref, deg_ref, y_ref, b1_ref, o_ref):
    agg = part_ref[0] + part_ref[1] + y_ref[...]
    deg = deg_ref[0, :, 0:1] + deg_ref[1, :, 0:1] + 1.0
    o_ref[...] = lax.rsqrt(deg) * agg + b1_ref[...]


def _final(part, deg, y, b1):
    return pl.pallas_call(
        _fin_body,
        grid=(NP // _RB,),
        in_specs=[pl.BlockSpec((NC, _RB, D), lambda i: (0, i, 0)),
                  pl.BlockSpec((NC, _RB, DEGW), lambda i: (0, i, 0)),
                  pl.BlockSpec((_RB, D), lambda i: (i, 0)),
                  pl.BlockSpec((1, D), lambda i: (0, 0))],
        out_specs=pl.BlockSpec((_RB, D), lambda i: (i, 0)),
        out_shape=jax.ShapeDtypeStruct((NP, D), _f32),
    )(part, deg, y, b1)


def kernel(edge_index, x_init, W0, b0, W1, b1):
    src = edge_index[0].astype(jnp.int32)
    dst = edge_index[1].astype(jnp.int32)
    xp = jnp.zeros((NP, D), _f32).at[:N].set(x_init)
    zrow = jnp.zeros((NP, D), _f32)
    zdeg = jnp.zeros((NP, DEGW), _f32)
    ones = jnp.ones((C, DEGW), _f32)
    b0r = b0.reshape(1, D)
    b1r = b1.reshape(1, D)

    deg = _deg(dst, zdeg, ones).reshape(NC, NP, DEGW)
    xw0 = _matmul(xp, W0)
    part0 = _agg(xw0, src, dst, zrow).reshape(NC, NP, D)
    y = _mid(part0, deg, W1, b0r)
    part1 = _agg(y, src, dst, zrow).reshape(NC, NP, D)
    return _final(part1, deg, y, b1r)[:N]
